# Initial kernel scaffold; baseline (speedup 1.0000x reference)
import jax
import jax.numpy as jnp
from jax.experimental import pallas as pl


def kernel(coords, hidden, edges, W1, b1, W2, b2, Wx1, bx1, Wx2, Wh1, bh1, Wh2, bh2):
    raise NotImplementedError("write your pallas kernel here")


# speedup vs baseline: 2.5833x; 2.5833x over previous
import functools

import jax
import jax.numpy as jnp
from jax import lax
from jax.experimental import pallas as pl
from jax.experimental.pallas import tpu as pltpu
from jax.experimental.pallas import tpu_sc as plsc

NC = 2
NS = 16
NW = NC * NS
CH = 80


def _silu(x):
    return x * jax.nn.sigmoid(x)


def _pre_body(h_ref, a_ref, b_ref, p_ref, q_ref):
    h = h_ref[...]
    p_ref[...] = jnp.dot(h, a_ref[...], preferred_element_type=jnp.float32)
    q_ref[...] = jnp.dot(h, b_ref[...], preferred_element_type=jnp.float32)


def _tc_precompute(hidden, A, B):
    N, H = hidden.shape
    M = A.shape[1]
    nb = 5
    Nb = N // nb
    return pl.pallas_call(
        _pre_body,
        grid=(nb,),
        in_specs=[
            pl.BlockSpec((Nb, H), lambda i: (i, 0)),
            pl.BlockSpec((H, M), lambda i: (0, 0)),
            pl.BlockSpec((H, M), lambda i: (0, 0)),
        ],
        out_specs=[
            pl.BlockSpec((Nb, M), lambda i: (i, 0)),
            pl.BlockSpec((Nb, M), lambda i: (i, 0)),
        ],
        out_shape=[jax.ShapeDtypeStruct((N, M), jnp.float32)] * 2,
    )(hidden, A, B)


def _sc_gather(P, Q, row, col, xs, ys, zs):
    N, M = P.shape
    E = row.shape[0]
    epw = E // NW
    nchunks = epw // CH
    mesh = plsc.VectorSubcoreMesh(
        core_axis_name="c", subcore_axis_name="s", num_cores=NC,
        num_subcores=NS)

    @functools.partial(
        pl.kernel,
        out_type=(
            jax.ShapeDtypeStruct((E, M), jnp.float32),
            jax.ShapeDtypeStruct((E, M), jnp.float32),
            jax.ShapeDtypeStruct((E * 8,), jnp.float32),
        ),
        mesh=mesh,
        scratch_types=[
            pltpu.VMEM((CH,), jnp.int32),
            pltpu.VMEM((CH,), jnp.int32),
            pltpu.VMEM((CH, M), jnp.float32),
            pltpu.VMEM((CH, M), jnp.float32),
            pltpu.VMEM((N,), jnp.float32),
            pltpu.VMEM((N,), jnp.float32),
            pltpu.VMEM((N,), jnp.float32),
            pltpu.VMEM((CH * 8,), jnp.float32),
            pltpu.SemaphoreType.DMA,
            pltpu.SemaphoreType.DMA,
        ],
        compiler_params=pltpu.CompilerParams(needs_layout_passes=False),
    )
    def k(p_hbm, q_hbm, row_hbm, col_hbm, xs_hbm, ys_hbm, zs_hbm,
          a_out, b_out, s_out,
          row_v, col_v, a_v, b_v, xs_v, ys_v, zs_v, s_v, sem_a, sem_b):
        wid = lax.axis_index("s") * NC + lax.axis_index("c")
        pltpu.sync_copy(xs_hbm, xs_v)
        pltpu.sync_copy(ys_hbm, ys_v)
        pltpu.sync_copy(zs_hbm, zs_v)
        base0 = wid * epw

        def chunk(i, carry):
            base = base0 + i * CH
            pltpu.sync_copy(row_hbm.at[pl.ds(base, CH)], row_v)
            pltpu.sync_copy(col_hbm.at[pl.ds(base, CH)], col_v)
            cp_a = pltpu.async_copy(p_hbm.at[row_v], a_v, sem_a)
            cp_b = pltpu.async_copy(q_hbm.at[col_v], b_v, sem_b)

            def sub(j, c2):
                r16 = row_v[pl.ds(j * 16, 16)]
                c16 = col_v[pl.ds(j * 16, 16)]
                dx = plsc.load_gather(xs_v, [r16]) - plsc.load_gather(xs_v, [c16])
                dy = plsc.load_gather(ys_v, [r16]) - plsc.load_gather(ys_v, [c16])
                dz = plsc.load_gather(zs_v, [r16]) - plsc.load_gather(zs_v, [c16])
                d2 = dx * dx + dy * dy + dz * dz
                fl = (lax.iota(jnp.int32, 16) + j * 16) * 8
                plsc.store_scatter(s_v, [fl], dx)
                plsc.store_scatter(s_v, [fl + 1], dy)
                plsc.store_scatter(s_v, [fl + 2], dz)
                plsc.store_scatter(s_v, [fl + 3], d2)
                return c2

            lax.fori_loop(0, CH // 16, sub, 0)
            cp_a.wait()
            cp_b.wait()
            pltpu.sync_copy(a_v, a_out.at[pl.ds(base, CH)])
            pltpu.sync_copy(b_v, b_out.at[pl.ds(base, CH)])
            pltpu.sync_copy(s_v, s_out.at[pl.ds(base * 8, CH * 8)])
            return carry

        lax.fori_loop(0, nchunks, chunk, 0)

    return k(P, Q, row, col, xs, ys, zs)


def _edge_body(a_ref, b_ref, sc_ref, w1r_ref, b1_ref, w2_ref, b2_ref,
               wx1_ref, bx1_ref, wx2t_ref, m_ref, s_ref):
    scal = sc_ref[...]
    d2 = scal[:, 3:4]
    e1 = a_ref[...] + b_ref[...] + d2 * w1r_ref[...] + b1_ref[...]
    m1 = _silu(e1)
    m = _silu(jnp.dot(m1, w2_ref[...], preferred_element_type=jnp.float32)
              + b2_ref[...])
    t = _silu(jnp.dot(m, wx1_ref[...], preferred_element_type=jnp.float32)
              + bx1_ref[...])
    w = jnp.tanh(jnp.sum(t * wx2t_ref[...], axis=1, keepdims=True))
    m_ref[...] = m
    trans = scal[:, 0:3] * w
    pad = jnp.zeros((trans.shape[0], 4), jnp.float32)
    s_ref[...] = jnp.concatenate([trans, jnp.ones_like(w), pad], axis=1)


def _tc_edge(Ar, Br, scal, w1r, b1, W2, b2, Wx1, bx1, Wx2):
    E, M = Ar.shape
    Eb = 2000
    nb = E // Eb
    rep = lambda i: (0, 0)
    return pl.pallas_call(
        _edge_body,
        grid=(nb,),
        in_specs=[
            pl.BlockSpec((Eb, M), lambda i: (i, 0)),
            pl.BlockSpec((Eb, M), lambda i: (i, 0)),
            pl.BlockSpec((Eb, 8), lambda i: (i, 0)),
            pl.BlockSpec((1, M), rep),
            pl.BlockSpec((1, M), rep),
            pl.BlockSpec((M, M), rep),
            pl.BlockSpec((1, M), rep),
            pl.BlockSpec((M, M), rep),
            pl.BlockSpec((1, M), rep),
            pl.BlockSpec((1, M), rep),
        ],
        out_specs=[
            pl.BlockSpec((Eb, M), lambda i: (i, 0)),
            pl.BlockSpec((Eb, 8), lambda i: (i, 0)),
        ],
        out_shape=[
            jax.ShapeDtypeStruct((E, M), jnp.float32),
            jax.ShapeDtypeStruct((E, 8), jnp.float32),
        ],
    )(Ar, Br, scal, w1r.reshape(1, M), b1.reshape(1, M), W2,
      b2.reshape(1, M), Wx1, bx1.reshape(1, M), Wx2.reshape(1, M))


def _node_body(nblocks, coords_ref, hid_ref, accm_ref, accs_ref,
               wh1a_ref, wh1b_ref, bh1_ref, wh2_ref, bh2_ref,
               co_ref, ho_ref, s1_ref, s2_ref, s1_acc, s2_acc):
    i = pl.program_id(0)
    magg = accm_ref[...]
    s = accs_ref[...]
    hid = hid_ref[...]
    deg = jnp.maximum(s[:, 3:4], 1.0)
    co_ref[...] = coords_ref[...] + s[:, 0:3] / deg
    h1 = _silu(jnp.dot(hid, wh1a_ref[...], preferred_element_type=jnp.float32)
               + jnp.dot(magg, wh1b_ref[...],
                         preferred_element_type=jnp.float32)
               + bh1_ref[...])
    h_out = hid + jnp.dot(h1, wh2_ref[...],
                          preferred_element_type=jnp.float32) + bh2_ref[...]
    ho_ref[...] = h_out

    @pl.when(i == 0)
    def _():
        s1_acc[...] = jnp.zeros_like(s1_acc)
        s2_acc[...] = jnp.zeros_like(s2_acc)

    s1_acc[...] += jnp.sum(h_out, axis=0, keepdims=True)
    s2_acc[...] += jnp.sum(h_out * h_out).reshape(1, 1)

    @pl.when(i == nblocks - 1)
    def _():
        s1_ref[...] = s1_acc[...]
        s2_ref[...] = s2_acc[...]


def _tc_node(coords, hidden, accM, accS, Wh1a, Wh1b, bh1, Wh2, bh2):
    N, H = hidden.shape
    M = accM.shape[-1]
    nb = 5
    Nb = N // nb
    rep = lambda i: (0, 0)
    return pl.pallas_call(
        functools.partial(_node_body, nb),
        grid=(nb,),
        in_specs=[
            pl.BlockSpec((Nb, 3), lambda i: (i, 0)),
            pl.BlockSpec((Nb, H), lambda i: (i, 0)),
            pl.BlockSpec((Nb, M), lambda i: (i, 0)),
            pl.BlockSpec((Nb, 8), lambda i: (i, 0)),
            pl.BlockSpec((H, M), rep),
            pl.BlockSpec((M, M), rep),
            pl.BlockSpec((1, M), rep),
            pl.BlockSpec((M, H), rep),
            pl.BlockSpec((1, H), rep),
        ],
        out_specs=[
            pl.BlockSpec((Nb, 3), lambda i: (i, 0)),
            pl.BlockSpec((Nb, H), lambda i: (i, 0)),
            pl.BlockSpec((1, H), rep),
            pl.BlockSpec((1, 1), rep),
        ],
        out_shape=[
            jax.ShapeDtypeStruct((N, 3), jnp.float32),
            jax.ShapeDtypeStruct((N, H), jnp.float32),
            jax.ShapeDtypeStruct((1, H), jnp.float32),
            jax.ShapeDtypeStruct((1, 1), jnp.float32),
        ],
        scratch_shapes=[
            pltpu.VMEM((1, H), jnp.float32),
            pltpu.VMEM((1, 1), jnp.float32),
        ],
    )(coords, hidden, accM, accS, Wh1a, Wh1b, bh1.reshape(1, M), Wh2,
      bh2.reshape(1, H))


def _norm_body(N, ho_ref, s1_ref, s2_ref, out_ref):
    mu = s1_ref[...] / N
    ms = s2_ref[0, 0] / N - jnp.sum(mu * mu)
    inv = lax.rsqrt(ms + 1e-6)
    out_ref[...] = (ho_ref[...] - mu) * inv


def _tc_norm(h_out, S1, S2):
    N, H = h_out.shape
    nb = 5
    Nb = N // nb
    rep = lambda i: (0, 0)
    return pl.pallas_call(
        functools.partial(_norm_body, N),
        grid=(nb,),
        in_specs=[
            pl.BlockSpec((Nb, H), lambda i: (i, 0)),
            pl.BlockSpec((1, H), rep),
            pl.BlockSpec((1, 1), rep),
        ],
        out_specs=pl.BlockSpec((Nb, H), lambda i: (i, 0)),
        out_shape=jax.ShapeDtypeStruct((N, H), jnp.float32),
    )(h_out, S1, S2)


def kernel(coords, hidden, edges, W1, b1, W2, b2, Wx1, bx1, Wx2,
           Wh1, bh1, Wh2, bh2):
    N, H = hidden.shape
    M = W2.shape[0]
    E = edges.shape[1]

    A = W1[:H]
    B = W1[H:2 * H]
    w1r = W1[2 * H]
    row = edges[0]
    col = edges[1]
    xs = coords[:, 0]
    ys = coords[:, 1]
    zs = coords[:, 2]

    P, Q = _tc_precompute(hidden, A, B)
    Ar, Br, scal_flat = _sc_gather(P, Q, row, col, xs, ys, zs)
    scal = scal_flat.reshape(E, 8)
    m_e, s_e = _tc_edge(Ar, Br, scal, w1r, b1, W2, b2, Wx1, bx1, Wx2)
    accM = jax.ops.segment_sum(m_e, row, num_segments=N)
    accS = jax.ops.segment_sum(s_e, row, num_segments=N)
    coords_out, h_out, S1, S2 = _tc_node(
        coords, hidden, accM, accS, Wh1[:H], Wh1[H:], bh1, Wh2, bh2)
    h_norm = _tc_norm(h_out, S1, S2)
    return (coords_out, h_norm)


# speedup vs baseline: 2.7024x; 1.0461x over previous
import functools

import jax
import jax.numpy as jnp
from jax import lax
from jax.experimental import pallas as pl
from jax.experimental.pallas import tpu as pltpu
from jax.experimental.pallas import tpu_sc as plsc

NC = 2
NS = 16
NW = NC * NS
CH = 80


def _silu(x):
    return x * jax.nn.sigmoid(x)


def _pre_body(h_ref, a_ref, b_ref, p_ref, q_ref):
    h = h_ref[...]
    p_ref[...] = jnp.dot(h, a_ref[...], preferred_element_type=jnp.float32)
    q_ref[...] = jnp.dot(h, b_ref[...], preferred_element_type=jnp.float32)


def _tc_precompute(hidden, A, B):
    N, H = hidden.shape
    M = A.shape[1]
    nb = 5
    Nb = N // nb
    return pl.pallas_call(
        _pre_body,
        grid=(nb,),
        in_specs=[
            pl.BlockSpec((Nb, H), lambda i: (i, 0)),
            pl.BlockSpec((H, M), lambda i: (0, 0)),
            pl.BlockSpec((H, M), lambda i: (0, 0)),
        ],
        out_specs=[
            pl.BlockSpec((Nb, M), lambda i: (i, 0)),
            pl.BlockSpec((Nb, M), lambda i: (i, 0)),
        ],
        out_shape=[jax.ShapeDtypeStruct((N, M), jnp.float32)] * 2,
    )(hidden, A, B)


def _sc_gather(P, Q, row, col, xs, ys, zs):
    N, M = P.shape
    E = row.shape[0]
    epw = E // NW
    nch = epw // CH
    assert nch >= 5 and nch % 2 == 1
    mesh = plsc.VectorSubcoreMesh(
        core_axis_name="c", subcore_axis_name="s", num_cores=NC,
        num_subcores=NS)

    @functools.partial(
        pl.kernel,
        out_type=(
            jax.ShapeDtypeStruct((E, M), jnp.float32),
            jax.ShapeDtypeStruct((E, M), jnp.float32),
            jax.ShapeDtypeStruct((E * 8,), jnp.float32),
        ),
        mesh=mesh,
        scratch_types=[
            [pltpu.VMEM((CH,), jnp.int32)] * 2,
            [pltpu.VMEM((CH,), jnp.int32)] * 2,
            [pltpu.VMEM((CH, M), jnp.float32)] * 2,
            [pltpu.VMEM((CH, M), jnp.float32)] * 2,
            [pltpu.VMEM((CH * 8,), jnp.float32)] * 2,
            pltpu.VMEM((N,), jnp.float32),
            pltpu.VMEM((N,), jnp.float32),
            pltpu.VMEM((N,), jnp.float32),
            [pltpu.SemaphoreType.DMA] * 2,
            [pltpu.SemaphoreType.DMA] * 2,
            [pltpu.SemaphoreType.DMA] * 2,
        ],
        compiler_params=pltpu.CompilerParams(needs_layout_passes=False),
    )
    def k(p_hbm, q_hbm, row_hbm, col_hbm, xs_hbm, ys_hbm, zs_hbm,
          a_out, b_out, s_out,
          row_v, col_v, a_v, b_v, s_v, xs_v, ys_v, zs_v,
          sem_idx, sem_g, sem_w):
        wid = lax.axis_index("s") * NC + lax.axis_index("c")
        base0 = wid * epw
        pltpu.sync_copy(xs_hbm, xs_v)
        pltpu.sync_copy(ys_hbm, ys_v)
        pltpu.sync_copy(zs_hbm, zs_v)

        def issue_idx(c, b):
            base = base0 + c * CH
            pltpu.async_copy(row_hbm.at[pl.ds(base, CH)], row_v[b], sem_idx[b])
            pltpu.async_copy(col_hbm.at[pl.ds(base, CH)], col_v[b], sem_idx[b])

        def drain_w(c, b):
            base = base0 + c * CH
            pltpu.make_async_copy(a_v[b], a_out.at[pl.ds(base, CH)],
                                  sem_w[b]).wait()
            pltpu.make_async_copy(b_v[b], b_out.at[pl.ds(base, CH)],
                                  sem_w[b]).wait()
            pltpu.make_async_copy(s_v[b], s_out.at[pl.ds(base * 8, CH * 8)],
                                  sem_w[b]).wait()

        def do_chunk(c, b, drain, prefetch):
            base = base0 + c * CH
            if drain:
                drain_w(c, b)
            pltpu.make_async_copy(row_hbm.at[pl.ds(base, CH)], row_v[b],
                                  sem_idx[b]).wait()
            pltpu.make_async_copy(col_hbm.at[pl.ds(base, CH)], col_v[b],
                                  sem_idx[b]).wait()
            cp_a = pltpu.async_copy(p_hbm.at[row_v[b]], a_v[b], sem_g[b])
            cp_b = pltpu.async_copy(q_hbm.at[col_v[b]], b_v[b], sem_g[b])

            def sub(j, c2):
                r16 = row_v[b][pl.ds(j * 16, 16)]
                c16 = col_v[b][pl.ds(j * 16, 16)]
                dx = plsc.load_gather(xs_v, [r16]) - plsc.load_gather(xs_v, [c16])
                dy = plsc.load_gather(ys_v, [r16]) - plsc.load_gather(ys_v, [c16])
                dz = plsc.load_gather(zs_v, [r16]) - plsc.load_gather(zs_v, [c16])
                d2 = dx * dx + dy * dy + dz * dz
                fl = (lax.iota(jnp.int32, 16) + j * 16) * 8
                plsc.store_scatter(s_v[b], [fl], dx)
                plsc.store_scatter(s_v[b], [fl + 1], dy)
                plsc.store_scatter(s_v[b], [fl + 2], dz)
                plsc.store_scatter(s_v[b], [fl + 3], d2)
                return c2

            lax.fori_loop(0, CH // 16, sub, 0)
            cp_a.wait()
            cp_b.wait()
            pltpu.async_copy(a_v[b], a_out.at[pl.ds(base, CH)], sem_w[b])
            pltpu.async_copy(b_v[b], b_out.at[pl.ds(base, CH)], sem_w[b])
            pltpu.async_copy(s_v[b], s_out.at[pl.ds(base * 8, CH * 8)],
                             sem_w[b])
            if prefetch:
                issue_idx(c + 2, b)

        issue_idx(0, 0)
        issue_idx(1, 1)
        do_chunk(0, 0, drain=False, prefetch=True)
        do_chunk(1, 1, drain=False, prefetch=True)

        def pair(i, carry):
            c = 2 * i
            do_chunk(c, 0, drain=True, prefetch=True)
            do_chunk(c + 1, 1, drain=True, prefetch=True)
            return carry

        lax.fori_loop(1, (nch - 3) // 2, pair, 0)
        do_chunk(nch - 3, 0, drain=True, prefetch=True)
        do_chunk(nch - 2, 1, drain=True, prefetch=False)
        do_chunk(nch - 1, 0, drain=True, prefetch=False)
        drain_w(nch - 1, 0)
        drain_w(nch - 2, 1)

    return k(P, Q, row, col, xs, ys, zs)


def _edge_body(a_ref, b_ref, sc_ref, w1r_ref, b1_ref, w2_ref, b2_ref,
               wx1_ref, bx1_ref, wx2t_ref, m_ref, s_ref):
    scal = sc_ref[...]
    d2 = scal[:, 3:4]
    e1 = a_ref[...] + b_ref[...] + d2 * w1r_ref[...] + b1_ref[...]
    m1 = _silu(e1)
    m = _silu(jnp.dot(m1, w2_ref[...], preferred_element_type=jnp.float32)
              + b2_ref[...])
    t = _silu(jnp.dot(m, wx1_ref[...], preferred_element_type=jnp.float32)
              + bx1_ref[...])
    w = jnp.tanh(jnp.sum(t * wx2t_ref[...], axis=1, keepdims=True))
    m_ref[...] = m
    trans = scal[:, 0:3] * w
    pad = jnp.zeros((trans.shape[0], 4), jnp.float32)
    s_ref[...] = jnp.concatenate([trans, jnp.ones_like(w), pad], axis=1)


def _tc_edge(Ar, Br, scal, w1r, b1, W2, b2, Wx1, bx1, Wx2):
    E, M = Ar.shape
    Eb = 2000
    nb = E // Eb
    rep = lambda i: (0, 0)
    return pl.pallas_call(
        _edge_body,
        grid=(nb,),
        in_specs=[
            pl.BlockSpec((Eb, M), lambda i: (i, 0)),
            pl.BlockSpec((Eb, M), lambda i: (i, 0)),
            pl.BlockSpec((Eb, 8), lambda i: (i, 0)),
            pl.BlockSpec((1, M), rep),
            pl.BlockSpec((1, M), rep),
            pl.BlockSpec((M, M), rep),
            pl.BlockSpec((1, M), rep),
            pl.BlockSpec((M, M), rep),
            pl.BlockSpec((1, M), rep),
            pl.BlockSpec((1, M), rep),
        ],
        out_specs=[
            pl.BlockSpec((Eb, M), lambda i: (i, 0)),
            pl.BlockSpec((Eb, 8), lambda i: (i, 0)),
        ],
        out_shape=[
            jax.ShapeDtypeStruct((E, M), jnp.float32),
            jax.ShapeDtypeStruct((E, 8), jnp.float32),
        ],
    )(Ar, Br, scal, w1r.reshape(1, M), b1.reshape(1, M), W2,
      b2.reshape(1, M), Wx1, bx1.reshape(1, M), Wx2.reshape(1, M))


def _node_body(nblocks, coords_ref, hid_ref, accm_ref, accs_ref,
               wh1a_ref, wh1b_ref, bh1_ref, wh2_ref, bh2_ref,
               co_ref, ho_ref, s1_ref, s2_ref, s1_acc, s2_acc):
    i = pl.program_id(0)
    magg = accm_ref[...]
    s = accs_ref[...]
    hid = hid_ref[...]
    deg = jnp.maximum(s[:, 3:4], 1.0)
    co_ref[...] = coords_ref[...] + s[:, 0:3] / deg
    h1 = _silu(jnp.dot(hid, wh1a_ref[...], preferred_element_type=jnp.float32)
               + jnp.dot(magg, wh1b_ref[...],
                         preferred_element_type=jnp.float32)
               + bh1_ref[...])
    h_out = hid + jnp.dot(h1, wh2_ref[...],
                          preferred_element_type=jnp.float32) + bh2_ref[...]
    ho_ref[...] = h_out

    @pl.when(i == 0)
    def _():
        s1_acc[...] = jnp.zeros_like(s1_acc)
        s2_acc[...] = jnp.zeros_like(s2_acc)

    s1_acc[...] += jnp.sum(h_out, axis=0, keepdims=True)
    s2_acc[...] += jnp.sum(h_out * h_out).reshape(1, 1)

    @pl.when(i == nblocks - 1)
    def _():
        s1_ref[...] = s1_acc[...]
        s2_ref[...] = s2_acc[...]


def _tc_node(coords, hidden, accM, accS, Wh1a, Wh1b, bh1, Wh2, bh2):
    N, H = hidden.shape
    M = accM.shape[-1]
    nb = 5
    Nb = N // nb
    rep = lambda i: (0, 0)
    return pl.pallas_call(
        functools.partial(_node_body, nb),
        grid=(nb,),
        in_specs=[
            pl.BlockSpec((Nb, 3), lambda i: (i, 0)),
            pl.BlockSpec((Nb, H), lambda i: (i, 0)),
            pl.BlockSpec((Nb, M), lambda i: (i, 0)),
            pl.BlockSpec((Nb, 8), lambda i: (i, 0)),
            pl.BlockSpec((H, M), rep),
            pl.BlockSpec((M, M), rep),
            pl.BlockSpec((1, M), rep),
            pl.BlockSpec((M, H), rep),
            pl.BlockSpec((1, H), rep),
        ],
        out_specs=[
            pl.BlockSpec((Nb, 3), lambda i: (i, 0)),
            pl.BlockSpec((Nb, H), lambda i: (i, 0)),
            pl.BlockSpec((1, H), rep),
            pl.BlockSpec((1, 1), rep),
        ],
        out_shape=[
            jax.ShapeDtypeStruct((N, 3), jnp.float32),
            jax.ShapeDtypeStruct((N, H), jnp.float32),
            jax.ShapeDtypeStruct((1, H), jnp.float32),
            jax.ShapeDtypeStruct((1, 1), jnp.float32),
        ],
        scratch_shapes=[
            pltpu.VMEM((1, H), jnp.float32),
            pltpu.VMEM((1, 1), jnp.float32),
        ],
    )(coords, hidden, accM, accS, Wh1a, Wh1b, bh1.reshape(1, M), Wh2,
      bh2.reshape(1, H))


def _norm_body(N, ho_ref, s1_ref, s2_ref, out_ref):
    mu = s1_ref[...] / N
    ms = s2_ref[0, 0] / N - jnp.sum(mu * mu)
    inv = lax.rsqrt(ms + 1e-6)
    out_ref[...] = (ho_ref[...] - mu) * inv


def _tc_norm(h_out, S1, S2):
    N, H = h_out.shape
    nb = 5
    Nb = N // nb
    rep = lambda i: (0, 0)
    return pl.pallas_call(
        functools.partial(_norm_body, N),
        grid=(nb,),
        in_specs=[
            pl.BlockSpec((Nb, H), lambda i: (i, 0)),
            pl.BlockSpec((1, H), rep),
            pl.BlockSpec((1, 1), rep),
        ],
        out_specs=pl.BlockSpec((Nb, H), lambda i: (i, 0)),
        out_shape=jax.ShapeDtypeStruct((N, H), jnp.float32),
    )(h_out, S1, S2)


def kernel(coords, hidden, edges, W1, b1, W2, b2, Wx1, bx1, Wx2,
           Wh1, bh1, Wh2, bh2):
    N, H = hidden.shape
    M = W2.shape[0]
    E = edges.shape[1]

    A = W1[:H]
    B = W1[H:2 * H]
    w1r = W1[2 * H]
    row = edges[0]
    col = edges[1]
    xs = coords[:, 0]
    ys = coords[:, 1]
    zs = coords[:, 2]

    P, Q = _tc_precompute(hidden, A, B)
    Ar, Br, scal_flat = _sc_gather(P, Q, row, col, xs, ys, zs)
    scal = scal_flat.reshape(E, 8)
    m_e, s_e = _tc_edge(Ar, Br, scal, w1r, b1, W2, b2, Wx1, bx1, Wx2)
    accM = jax.ops.segment_sum(m_e, row, num_segments=N)
    accS = jax.ops.segment_sum(s_e, row, num_segments=N)
    coords_out, h_out, S1, S2 = _tc_node(
        coords, hidden, accM, accS, Wh1[:H], Wh1[H:], bh1, Wh2, bh2)
    h_norm = _tc_norm(h_out, S1, S2)
    return (coords_out, h_norm)
